# Initial kernel scaffold; baseline (speedup 1.0000x reference)
import jax
import jax.numpy as jnp
from jax.experimental import pallas as pl


def kernel(h, coord, edge_index, coord_diff, edge_attr, node_mask, edge_mask, W1, b1, W2, b2, W3):
    raise NotImplementedError("write your pallas kernel here")


# speedup vs baseline: 1.4741x; 1.4741x over previous
import dataclasses
import functools

import jax
import jax.numpy as jnp
from jax import lax
from jax.experimental import pallas as pl
from jax.experimental.pallas import tpu as pltpu
from jax.experimental.pallas import tpu_sc as plsc

N_NODES = 10000
N_EDGES = 320000
H = 128
EDGES_IN_D = 4
NORM = 100.0

EP = 327680
NP = 10240
GW = 128
CB = 2048


def _tables_body(h_ref, w1at_ref, w1bt_ref, ha_ref, hb_ref):
    hv = h_ref[...]
    ha_ref[...] = jnp.dot(hv, w1at_ref[...], preferred_element_type=jnp.float32)
    hb_ref[...] = jnp.dot(hv, w1bt_ref[...], preferred_element_type=jnp.float32)


def _make_tables(h, w1at, w1bt):
    return pl.pallas_call(
        _tables_body,
        out_shape=(
            jax.ShapeDtypeStruct((N_NODES, H), jnp.float32),
            jax.ShapeDtypeStruct((N_NODES, H), jnp.float32),
        ),
    )(h, w1at, w1bt)


def _gather_sc(ha, hb, row2d, col2d):
    mesh = plsc.VectorSubcoreMesh(core_axis_name="c", subcore_axis_name="s")

    @functools.partial(
        pl.kernel,
        out_type=(
            jax.ShapeDtypeStruct((EP, H), jnp.float32),
            jax.ShapeDtypeStruct((EP, H), jnp.float32),
        ),
        mesh=mesh,
    )
    def gk(ha_hbm, hb_hbm, row_hbm, col_hbm, ga_hbm, gb_hbm):
        def body(ri_vmem, ci_vmem, ga_vmem, gb_vmem):
            pltpu.sync_copy(ha_hbm.at[ri_vmem.at[0]], ga_vmem)
            pltpu.sync_copy(hb_hbm.at[ci_vmem.at[0]], gb_vmem)

        pltpu.emit_pipeline(
            body,
            grid=(EP // GW,),
            in_specs=[
                pl.BlockSpec((1, GW), lambda i: (0, i)),
                pl.BlockSpec((1, GW), lambda i: (0, i)),
            ],
            out_specs=[
                pl.BlockSpec((GW, H), lambda i: (i, 0)),
                pl.BlockSpec((GW, H), lambda i: (i, 0)),
            ],
            core_axis_name=("c", "s"),
            dimension_semantics=(pltpu.PARALLEL,),
        )(row_hbm, col_hbm, ga_hbm, gb_hbm)

    return gk(ha, hb, row2d, col2d)


def _mlp_body(ga_ref, gb_ref, ea_ref, cd_ref, em_ref,
              w1ct_ref, b1_ref, w2t_ref, b2_ref, w3t_ref, out_ref):
    pre1 = (ga_ref[...] + gb_ref[...] + b1_ref[...]
            + jnp.dot(ea_ref[...], w1ct_ref[...],
                      preferred_element_type=jnp.float32))
    x1 = pre1 * jax.nn.sigmoid(pre1)
    pre2 = jnp.dot(x1, w2t_ref[...],
                   preferred_element_type=jnp.float32) + b2_ref[...]
    x2 = pre2 * jax.nn.sigmoid(pre2)
    m = jnp.dot(x2, w3t_ref[...], preferred_element_type=jnp.float32)
    out_ref[...] = cd_ref[...] * (m * (1.0 / NORM)) * em_ref[...]


def _edge_mlp(ga, gb, ea, cd4, em, w1ct, b1, w2t, b2, w3t):
    nblk = EP // CB
    full = lambda i: (0, 0)
    return pl.pallas_call(
        _mlp_body,
        grid=(nblk,),
        in_specs=[
            pl.BlockSpec((CB, H), lambda i: (i, 0)),
            pl.BlockSpec((CB, H), lambda i: (i, 0)),
            pl.BlockSpec((CB, EDGES_IN_D), lambda i: (i, 0)),
            pl.BlockSpec((CB, 4), lambda i: (i, 0)),
            pl.BlockSpec((CB, 1), lambda i: (i, 0)),
            pl.BlockSpec((EDGES_IN_D, H), full),
            pl.BlockSpec((1, H), full),
            pl.BlockSpec((H, H), full),
            pl.BlockSpec((1, H), full),
            pl.BlockSpec((H, 1), full),
        ],
        out_specs=pl.BlockSpec((CB, 4), lambda i: (i, 0)),
        out_shape=jax.ShapeDtypeStruct((EP, 4), jnp.float32),
    )(ga, gb, ea, cd4, em, w1ct, b1, w2t, b2, w3t)


def _sc_compiler_params():
    cp = pltpu.CompilerParams()
    if "needs_layout_passes" in pltpu.CompilerParams.__dataclass_fields__:
        cp = dataclasses.replace(cp, needs_layout_passes=False)
    return cp


def _scatter_sc(row, trans_flat, zeros_flat):
    mesh = plsc.VectorSubcoreMesh(core_axis_name="c", subcore_axis_name="s")
    per_worker = EP // 32
    nchunks = per_worker // GW

    @functools.partial(
        pl.kernel,
        out_type=jax.ShapeDtypeStruct((32, NP * 4), jnp.float32),
        mesh=mesh,
        scratch_types=[
            pltpu.VMEM((NP * 4,), jnp.float32),
            pltpu.VMEM((GW,), jnp.int32),
            pltpu.VMEM((GW * 4,), jnp.float32),
        ],
        compiler_params=_sc_compiler_params(),
    )
    def sk(row_hbm, trans_hbm, zero_hbm, parts_hbm, acc_v, idx_v, dat_v):
        cid = lax.axis_index("c")
        sid = lax.axis_index("s")
        wid = cid * 16 + sid
        base = wid * per_worker

        pltpu.sync_copy(zero_hbm, acc_v)
        sel = lax.iota(jnp.int32, 16) // 4
        lane4 = lax.iota(jnp.int32, 16) % 4

        @pl.loop(0, nchunks)
        def _(c):
            off = base + c * GW
            pltpu.sync_copy(row_hbm.at[pl.ds(off, GW)], idx_v)
            pltpu.sync_copy(trans_hbm.at[pl.ds(off * 4, GW * 4)], dat_v)
            for g in range(GW // 4):
                rows = plsc.load_gather(idx_v, [sel + g * 4])
                tgt = rows * 4 + lane4
                vals = dat_v[pl.ds(g * 16, 16)]
                plsc.addupdate_scatter(acc_v, [tgt], vals)

        pltpu.sync_copy(acc_v, parts_hbm.at[wid])

    return sk(row, trans_flat, zeros_flat)


def _combine_body(parts_ref, cm_ref, nm_ref, out_ref):
    out_ref[...] = (cm_ref[...] + jnp.sum(parts_ref[...], axis=0)) * nm_ref[...]


def _combine(parts, cm, nm):
    return pl.pallas_call(
        _combine_body,
        out_shape=jax.ShapeDtypeStruct((NP * 4 // 128, 128), jnp.float32),
    )(parts, cm, nm)


def kernel(h, coord, edge_index, coord_diff, edge_attr, node_mask, edge_mask,
           W1, b1, W2, b2, W3):
    f32 = jnp.float32
    row = edge_index[0].astype(jnp.int32)
    col = edge_index[1].astype(jnp.int32)

    w1t = W1.T
    w1at = w1t[:H]
    w1bt = w1t[H:2 * H]
    w1ct = w1t[2 * H:]
    b1r = b1.reshape(1, H)
    b2r = b2.reshape(1, H)
    w2t = W2.T
    w3t = W3.T

    pe = EP - N_EDGES
    rowp = jnp.pad(row, (0, pe))
    colp = jnp.pad(col, (0, pe))
    eap = jnp.pad(edge_attr, ((0, pe), (0, 0)))
    cd4p = jnp.pad(coord_diff, ((0, pe), (0, 1)))
    emp = jnp.pad(edge_mask, ((0, pe), (0, 0)))

    ha, hb = _make_tables(h, w1at, w1bt)
    ga, gb = _gather_sc(ha, hb, rowp.reshape(1, EP), colp.reshape(1, EP))
    trans4 = _edge_mlp(ga, gb, eap, cd4p, emp, w1ct, b1r, w2t, b2r, w3t)

    zeros_flat = jnp.zeros((NP * 4,), dtype=f32)
    parts = _scatter_sc(rowp, trans4.reshape(EP * 4), zeros_flat)

    cm = jnp.pad(coord, ((0, NP - N_NODES), (0, 1))).reshape(NP * 4 // 128, 128)
    nm = jnp.pad(jnp.broadcast_to(node_mask, (N_NODES, 4)),
                 ((0, NP - N_NODES), (0, 0))).reshape(NP * 4 // 128, 128)
    out = _combine(parts.reshape(32, NP * 4 // 128, 128), cm, nm)
    return out.reshape(NP, 4)[:N_NODES, :3]
